# baseline (speedup 1.0000x reference)
import jax
import jax.numpy as jnp
from jax import lax
from jax.experimental import pallas as pl
from jax.experimental.pallas import tpu as pltpu
from jax.experimental.pallas import tpu_sc as plsc

NUM_ACTIONS = 16
D = 128
NUM_ARGS = 3
L_TOTAL = 819200

NC = 2
NS = 16
NW = NC * NS
CHUNK = 256
GATHER = 128


def _build_combined(table):
    def body(t_ref, out_ref):
        t = t_ref[...]
        tb = jnp.broadcast_to(t[None], (NUM_ARGS, NUM_ACTIONS, D - 1))
        g = lax.broadcasted_iota(jnp.int32, (NUM_ARGS, NUM_ACTIONS, 1), 0).astype(jnp.float32)
        comb = jnp.concatenate([tb, g], axis=2)
        out_ref[...] = comb.reshape(NUM_ARGS * NUM_ACTIONS, D)

    return pl.pallas_call(
        body,
        out_shape=jax.ShapeDtypeStruct((NUM_ACTIONS * NUM_ARGS, D), jnp.float32),
    )(table)


def _sc_lookup_body(comb_hbm, act_hbm, arg_hbm, out_hbm,
                    a_v, g_v, idx_v, rows0, rows1, sem_g, sem_o):
    rows_per_w = L_TOTAL // NW
    n_chunks = rows_per_w // CHUNK
    wid = lax.axis_index("s") * NC + lax.axis_index("c")
    w_base = wid * rows_per_w
    rows = [rows0, rows1]

    def do_chunk(c, b, wait_prev_out):
        base = w_base + c * CHUNK
        pltpu.sync_copy(act_hbm.at[pl.ds(base, CHUNK)], a_v)
        pltpu.sync_copy(arg_hbm.at[pl.ds(base, CHUNK)], g_v)
        for i in range(CHUNK // 16):
            a = a_v[pl.ds(i * 16, 16)]
            g = g_v[pl.ds(i * 16, 16)]
            idx_v[i // 8, pl.ds((i % 8) * 16, 16)] = g * NUM_ACTIONS + a
        copies = []
        for j in range(CHUNK // GATHER):
            copies.append(
                pltpu.async_copy(
                    comb_hbm.at[idx_v.at[j]],
                    rows[b].at[pl.ds(j * GATHER, GATHER)],
                    sem_g,
                ))
        for cp in copies:
            cp.wait()
        if wait_prev_out:
            pltpu.make_async_copy(
                rows[1 - b], out_hbm.at[pl.ds(w_base, CHUNK)], sem_o).wait()
        pltpu.async_copy(rows[b], out_hbm.at[pl.ds(base, CHUNK)], sem_o)

    do_chunk(0, 0, False)

    def pair(k, carry):
        c = 1 + 2 * k
        do_chunk(c, 1, True)
        do_chunk(c + 1, 0, True)
        return carry

    lax.fori_loop(0, (n_chunks - 2) // 2, pair, 0, unroll=False)
    do_chunk(n_chunks - 1, 1, True)
    pltpu.make_async_copy(
        rows[1], out_hbm.at[pl.ds(w_base, CHUNK)], sem_o).wait()


@jax.jit
def kernel(actions, arguments, table):
    comb = _build_combined(table.astype(jnp.float32))
    act = actions.astype(jnp.int32)
    arg = arguments.astype(jnp.int32)

    mesh = plsc.VectorSubcoreMesh(core_axis_name="c", subcore_axis_name="s")
    lookup = pl.kernel(
        _sc_lookup_body,
        out_type=jax.ShapeDtypeStruct((L_TOTAL, D), jnp.float32),
        mesh=mesh,
        scratch_types=[
            pltpu.VMEM((CHUNK,), jnp.int32),
            pltpu.VMEM((CHUNK,), jnp.int32),
            pltpu.VMEM((CHUNK // GATHER, GATHER), jnp.int32),
            pltpu.VMEM((CHUNK, D), jnp.float32),
            pltpu.VMEM((CHUNK, D), jnp.float32),
            pltpu.SemaphoreType.DMA,
            pltpu.SemaphoreType.DMA,
        ],
    )
    return lookup(comb, act, arg)


# speedup vs baseline: 1.0072x; 1.0072x over previous
import jax
import jax.numpy as jnp
from jax import lax
from jax.experimental import pallas as pl
from jax.experimental.pallas import tpu as pltpu
from jax.experimental.pallas import tpu_sc as plsc

NUM_ACTIONS = 16
D = 128
NUM_ARGS = 3
L_TOTAL = 819200

NC = 2
NS = 16
NW = NC * NS
CHUNK = 256
GATHER = 128


def _build_combined(table):
    def body(t_ref, out_ref):
        t = t_ref[...]
        tb = jnp.broadcast_to(t[None], (NUM_ARGS, NUM_ACTIONS, D - 1))
        g = lax.broadcasted_iota(jnp.int32, (NUM_ARGS, NUM_ACTIONS, 1), 0).astype(jnp.float32)
        comb = jnp.concatenate([tb, g], axis=2)
        out_ref[...] = comb.reshape(NUM_ARGS * NUM_ACTIONS, D)

    return pl.pallas_call(
        body,
        out_shape=jax.ShapeDtypeStruct((NUM_ACTIONS * NUM_ARGS, D), jnp.float32),
    )(table)


def _sc_lookup_body(comb_hbm, act_hbm, arg_hbm, out_hbm,
                    a_v, g_v, idx0, idx1, rows0, rows1,
                    sem_g0, sem_g1, sem_o0, sem_o1):
    rows_per_w = L_TOTAL // NW
    n_chunks = rows_per_w // CHUNK
    wid = lax.axis_index("s") * NC + lax.axis_index("c")
    w_base = wid * rows_per_w
    rows = [rows0, rows1]
    idx = [idx0, idx1]
    sem_g = [sem_g0, sem_g1]
    sem_o = [sem_o0, sem_o1]

    def fire_gathers(c, b):
        base = w_base + c * CHUNK
        pltpu.sync_copy(act_hbm.at[pl.ds(base, CHUNK)], a_v)
        pltpu.sync_copy(arg_hbm.at[pl.ds(base, CHUNK)], g_v)
        for i in range(CHUNK // 16):
            a = a_v[pl.ds(i * 16, 16)]
            g = g_v[pl.ds(i * 16, 16)]
            idx[b][i // 8, pl.ds((i % 8) * 16, 16)] = g * NUM_ACTIONS + a
        for j in range(CHUNK // GATHER):
            pltpu.async_copy(
                comb_hbm.at[idx[b].at[j]],
                rows[b].at[pl.ds(j * GATHER, GATHER)],
                sem_g[b],
            )

    def drain_gathers(b):
        for j in range(CHUNK // GATHER):
            pltpu.make_async_copy(
                comb_hbm.at[idx[b].at[j]],
                rows[b].at[pl.ds(j * GATHER, GATHER)],
                sem_g[b],
            ).wait()

    def drain_out(b):
        pltpu.make_async_copy(
            rows[b], out_hbm.at[pl.ds(w_base, CHUNK)], sem_o[b]).wait()

    def step(c, b, drain_prev_out, prefetch):
        nb = 1 - b
        if prefetch:
            if drain_prev_out:
                drain_out(nb)
            fire_gathers(c + 1, nb)
        drain_gathers(b)
        pltpu.async_copy(
            rows[b], out_hbm.at[pl.ds(w_base + c * CHUNK, CHUNK)], sem_o[b])

    fire_gathers(0, 0)
    step(0, 0, False, True)

    def pair(k, carry):
        c = 1 + 2 * k
        step(c, 1, True, True)
        step(c + 1, 0, True, True)
        return carry

    lax.fori_loop(0, (n_chunks - 2) // 2, pair, 0, unroll=False)
    step(n_chunks - 1, 1, False, False)
    drain_out(0)
    drain_out(1)


@jax.jit
def kernel(actions, arguments, table):
    comb = _build_combined(table.astype(jnp.float32))
    act = actions.astype(jnp.int32)
    arg = arguments.astype(jnp.int32)

    mesh = plsc.VectorSubcoreMesh(core_axis_name="c", subcore_axis_name="s")
    lookup = pl.kernel(
        _sc_lookup_body,
        out_type=jax.ShapeDtypeStruct((L_TOTAL, D), jnp.float32),
        mesh=mesh,
        scratch_types=[
            pltpu.VMEM((CHUNK,), jnp.int32),
            pltpu.VMEM((CHUNK,), jnp.int32),
            pltpu.VMEM((CHUNK // GATHER, GATHER), jnp.int32),
            pltpu.VMEM((CHUNK // GATHER, GATHER), jnp.int32),
            pltpu.VMEM((CHUNK, D), jnp.float32),
            pltpu.VMEM((CHUNK, D), jnp.float32),
            pltpu.SemaphoreType.DMA,
            pltpu.SemaphoreType.DMA,
            pltpu.SemaphoreType.DMA,
            pltpu.SemaphoreType.DMA,
        ],
    )
    return lookup(comb, act, arg)


# speedup vs baseline: 3.1155x; 3.0931x over previous
import jax
import jax.numpy as jnp
from jax import lax
from jax.experimental import pallas as pl
from jax.experimental.pallas import tpu as pltpu
from jax.experimental.pallas import tpu_sc as plsc

NUM_ACTIONS = 16
D = 128
NUM_ARGS = 3
L_TOTAL = 819200

NC = 2
NS = 16
NW = NC * NS
CHUNK = 256
GATHER = 128


def _build_combined(table):
    def body(t_ref, out_ref):
        t = t_ref[...]
        tb = jnp.broadcast_to(t[None], (NUM_ARGS, NUM_ACTIONS, D - 1))
        g = lax.broadcasted_iota(jnp.int32, (NUM_ARGS, NUM_ACTIONS, 1), 0).astype(jnp.float32)
        comb = jnp.concatenate([tb, g], axis=2)
        comb = comb.reshape(NUM_ACTIONS * NUM_ARGS, D)
        rep = jnp.broadcast_to(comb[None], (NW, NUM_ACTIONS * NUM_ARGS, D))
        out_ref[...] = rep.reshape(NW * NUM_ACTIONS * NUM_ARGS, D)

    return pl.pallas_call(
        body,
        out_shape=jax.ShapeDtypeStruct((NW * NUM_ACTIONS * NUM_ARGS, D), jnp.float32),
    )(table)


def _sc_lookup_body(comb_hbm, act_hbm, arg_hbm, out_hbm,
                    a_v, g_v, idx0, idx1, rows0, rows1,
                    sem_g0, sem_g1, sem_o0, sem_o1):
    rows_per_w = L_TOTAL // NW
    n_chunks = rows_per_w // CHUNK
    wid = lax.axis_index("s") * NC + lax.axis_index("c")
    w_base = wid * rows_per_w
    rows = [rows0, rows1]
    idx = [idx0, idx1]
    sem_g = [sem_g0, sem_g1]
    sem_o = [sem_o0, sem_o1]
    tbl_base = wid * (NUM_ACTIONS * NUM_ARGS)

    def fire_gathers(c, b):
        base = w_base + c * CHUNK
        pltpu.sync_copy(act_hbm.at[pl.ds(base, CHUNK)], a_v)
        pltpu.sync_copy(arg_hbm.at[pl.ds(base, CHUNK)], g_v)
        for i in range(CHUNK // 16):
            a = a_v[pl.ds(i * 16, 16)]
            g = g_v[pl.ds(i * 16, 16)]
            idx[b][i // 8, pl.ds((i % 8) * 16, 16)] = g * NUM_ACTIONS + a + tbl_base
        for j in range(CHUNK // GATHER):
            pltpu.async_copy(
                comb_hbm.at[idx[b].at[j]],
                rows[b].at[pl.ds(j * GATHER, GATHER)],
                sem_g[b],
            )

    def drain_gathers(b):
        for j in range(CHUNK // GATHER):
            pltpu.make_async_copy(
                comb_hbm.at[idx[b].at[j]],
                rows[b].at[pl.ds(j * GATHER, GATHER)],
                sem_g[b],
            ).wait()

    def drain_out(b):
        pltpu.make_async_copy(
            rows[b], out_hbm.at[pl.ds(w_base, CHUNK)], sem_o[b]).wait()

    def step(c, b, drain_prev_out, prefetch):
        nb = 1 - b
        if prefetch:
            if drain_prev_out:
                drain_out(nb)
            fire_gathers(c + 1, nb)
        drain_gathers(b)
        pltpu.async_copy(
            rows[b], out_hbm.at[pl.ds(w_base + c * CHUNK, CHUNK)], sem_o[b])

    fire_gathers(0, 0)
    step(0, 0, False, True)

    def pair(k, carry):
        c = 1 + 2 * k
        step(c, 1, True, True)
        step(c + 1, 0, True, True)
        return carry

    lax.fori_loop(0, (n_chunks - 2) // 2, pair, 0, unroll=False)
    step(n_chunks - 1, 1, False, False)
    drain_out(0)
    drain_out(1)


@jax.jit
def kernel(actions, arguments, table):
    comb = _build_combined(table.astype(jnp.float32))
    act = actions.astype(jnp.int32)
    arg = arguments.astype(jnp.int32)

    mesh = plsc.VectorSubcoreMesh(core_axis_name="c", subcore_axis_name="s")
    lookup = pl.kernel(
        _sc_lookup_body,
        out_type=jax.ShapeDtypeStruct((L_TOTAL, D), jnp.float32),
        mesh=mesh,
        scratch_types=[
            pltpu.VMEM((CHUNK,), jnp.int32),
            pltpu.VMEM((CHUNK,), jnp.int32),
            pltpu.VMEM((CHUNK // GATHER, GATHER), jnp.int32),
            pltpu.VMEM((CHUNK // GATHER, GATHER), jnp.int32),
            pltpu.VMEM((CHUNK, D), jnp.float32),
            pltpu.VMEM((CHUNK, D), jnp.float32),
            pltpu.SemaphoreType.DMA,
            pltpu.SemaphoreType.DMA,
            pltpu.SemaphoreType.DMA,
            pltpu.SemaphoreType.DMA,
        ],
    )
    return lookup(comb, act, arg)
